# baseline (speedup 1.0000x reference)
import jax
import jax.numpy as jnp
from jax.experimental import pallas as pl
from jax.experimental.pallas import tpu as pltpu

B, S, H = 2, 2048, 1024
NH, NKV, HD = 16, 4, 64
I, E, K = 512, 8, 2
GRP = NH // NKV
EPS = 1e-6
T = B * S

BLK_T = 512
BLK_Q = 512
BLK_K = 512
QROWS = GRP * BLK_Q
NEG = -1e30


def _rms(x, w):
    var = jnp.mean(x * x, axis=-1, keepdims=True)
    return x * jax.lax.rsqrt(var + EPS) * w


def _pre_kernel(hs_ref, ln1_ref, wq_ref, wk_ref, wv_ref, q_ref, k_ref, v_ref):
    x = hs_ref[...]
    xn = _rms(x, ln1_ref[...]).astype(jnp.bfloat16)
    q = jax.lax.dot(xn, wq_ref[...],
                    preferred_element_type=jnp.float32).astype(jnp.bfloat16)
    k = jax.lax.dot(xn, wk_ref[...],
                    preferred_element_type=jnp.float32).astype(jnp.bfloat16)
    v = jax.lax.dot(xn, wv_ref[...], preferred_element_type=jnp.float32)
    for h in range(NH):
        q_ref[h] = q[:, h * HD:(h + 1) * HD]
    lane = jax.lax.broadcasted_iota(jnp.int32, (BLK_T, 128), 1)
    ones_col = (lane == HD).astype(jnp.float32)
    for h in range(NKV):
        k_ref[h] = k[:, h * HD:(h + 1) * HD]
        vh = jnp.pad(v[:, h * HD:(h + 1) * HD], ((0, 0), (0, 128 - HD)))
        v_ref[h] = (vh + ones_col).astype(jnp.bfloat16)


def _rope(x, cos, sin):
    x1 = x[:, :HD // 2]
    x2 = x[:, HD // 2:]
    rot = jnp.concatenate([-x2, x1], axis=-1)
    return x * cos + rot * sin


def _flash_kernel(q_ref, k_ref, v_ref, cq_ref, sq_ref, ck_ref, sk_ref,
                  mask_ref, o_ref, acc_ref, m_ref, l_ref):
    iq = pl.program_id(2)
    ik = pl.program_id(3)

    @pl.when(ik == 0)
    def _():
        m_ref[...] = jnp.full_like(m_ref, NEG)
        l_ref[...] = jnp.zeros_like(l_ref)
        acc_ref[...] = jnp.zeros_like(acc_ref)

    @pl.when(ik <= iq)
    def _():
        q = _rope(q_ref[...].reshape(QROWS, HD).astype(jnp.float32),
                  cq_ref[0], sq_ref[0])
        k = _rope(k_ref[0].astype(jnp.float32), ck_ref[...], sk_ref[...])
        qb = (q * (HD ** -0.5)).astype(jnp.bfloat16)
        kb = k.astype(jnp.bfloat16)
        s = jax.lax.dot_general(qb, kb, (((1,), (1,)), ((), ())),
                                preferred_element_type=jnp.float32)
        ind = (ik == iq).astype(jnp.float32)
        s = s + mask_ref[...] * ind

        m_prev = m_ref[...]
        l_prev = l_ref[...]
        m_cur = jnp.max(s, axis=1, keepdims=True)
        m_next = jnp.maximum(m_prev, m_cur)
        alpha = jnp.exp(m_prev - m_next)
        p = jnp.exp(s - m_next[:, :1])
        pv = jax.lax.dot(p.astype(jnp.bfloat16), v_ref[0],
                         preferred_element_type=jnp.float32)
        l_ref[...] = alpha * l_prev + pv[:, HD:HD + 1]
        m_ref[...] = m_next
        acc_ref[...] = acc_ref[...] * alpha[:, :1] + pv

    @pl.when(ik == iq)
    def _():
        out = acc_ref[:, :HD] / l_ref[:, :1]
        o_ref[...] = out.reshape(GRP, BLK_Q, HD)


def _post_kernel(attn_ref, hs_ref, ln2_ref, wo_ref, rw_ref,
                 res2_ref, xn2_ref, comb_ref):
    a = jnp.concatenate([attn_ref[h] for h in range(NH)], axis=1)
    o = jax.lax.dot(a.astype(jnp.bfloat16), wo_ref[...],
                    preferred_element_type=jnp.float32)
    res2 = hs_ref[...] + o
    res2_ref[...] = res2
    xn = _rms(res2, ln2_ref[...])
    xnb = xn.astype(jnp.bfloat16)
    xn2_ref[...] = xnb
    logits = jax.lax.dot(xnb, rw_ref[...], preferred_element_type=jnp.float32)
    lane = jax.lax.broadcasted_iota(jnp.int32, (BLK_T, 128), 1)
    valid = lane < E
    lg = jnp.where(valid, logits, NEG)
    mx = jnp.max(lg, axis=1, keepdims=True)
    ex = jnp.where(valid, jnp.exp(lg - mx), 0.0)
    denom = jnp.sum(ex, axis=1, keepdims=True)
    sc = ex / denom
    m1 = jnp.max(sc, axis=1, keepdims=True)
    idx1 = jnp.min(jnp.where(sc == m1, lane, 128), axis=1, keepdims=True)
    is1 = lane == idx1
    sc2 = jnp.where(is1, -1.0, sc)
    m2 = jnp.max(sc2, axis=1, keepdims=True)
    idx2 = jnp.min(jnp.where(sc2 == m2, lane, 128), axis=1, keepdims=True)
    is2 = lane == idx2
    wsum = m1 + m2
    comb_ref[...] = jnp.where(is1, m1 / wsum, jnp.where(is2, m2 / wsum, 0.0))


def _moe_kernel(x_ref, comb_ref, res2_ref, wg_ref, wu_ref, wd_ref, y_ref):
    e = pl.program_id(1)
    x = x_ref[...]
    gate = jax.lax.dot(x, wg_ref[0], preferred_element_type=jnp.float32)
    up = jax.lax.dot(x, wu_ref[0], preferred_element_type=jnp.float32)
    act = (gate * jax.lax.logistic(gate)) * up
    lane = jax.lax.broadcasted_iota(jnp.int32, (BLK_T, 128), 1)
    c = jnp.sum(jnp.where(lane == e, comb_ref[...], 0.0), axis=1,
                keepdims=True)
    actb = (act * c).astype(jnp.bfloat16)
    out_e = jax.lax.dot(actb, wd_ref[0], preferred_element_type=jnp.float32)

    @pl.when(e == 0)
    def _():
        y_ref[...] = res2_ref[...] + out_e

    @pl.when(e > 0)
    def _():
        y_ref[...] = y_ref[...] + out_e


def _build(hidden_states, cos, sin, Wq, Wk, Wv, Wo, ln1_w, ln2_w,
           router_w, Wg, Wu, Wd):
    hs = hidden_states.reshape(T, H)
    ln1 = ln1_w.reshape(1, H)
    ln2 = ln2_w.reshape(1, H)
    wq_t = Wq.T.astype(jnp.bfloat16)
    wk_t = Wk.T.astype(jnp.bfloat16)
    wv_t = Wv.T.astype(jnp.bfloat16)
    wo_t = Wo.T.astype(jnp.bfloat16)
    rw_pad = jnp.zeros((128, H), jnp.float32).at[:E].set(router_w)
    rw_t = rw_pad.T.astype(jnp.bfloat16)

    n_t = T // BLK_T
    q, k, v = pl.pallas_call(
        _pre_kernel,
        grid=(n_t,),
        in_specs=[
            pl.BlockSpec((BLK_T, H), lambda i: (i, 0)),
            pl.BlockSpec((1, H), lambda i: (0, 0)),
            pl.BlockSpec((H, NH * HD), lambda i: (0, 0)),
            pl.BlockSpec((H, NKV * HD), lambda i: (0, 0)),
            pl.BlockSpec((H, NKV * HD), lambda i: (0, 0)),
        ],
        out_specs=[
            pl.BlockSpec((NH, BLK_T, HD), lambda i: (0, i, 0)),
            pl.BlockSpec((NKV, BLK_T, HD), lambda i: (0, i, 0)),
            pl.BlockSpec((NKV, BLK_T, 128), lambda i: (0, i, 0)),
        ],
        out_shape=[
            jax.ShapeDtypeStruct((NH, T, HD), jnp.bfloat16),
            jax.ShapeDtypeStruct((NKV, T, HD), jnp.bfloat16),
            jax.ShapeDtypeStruct((NKV, T, 128), jnp.bfloat16),
        ],
    )(hs, ln1, wq_t, wk_t, wv_t)

    n_q = S // BLK_Q
    n_k = S // BLK_K
    cos_q = jnp.tile(cos.reshape(n_q, 1, BLK_Q, HD), (1, GRP, 1, 1)) \
        .reshape(n_q, QROWS, HD)
    sin_q = jnp.tile(sin.reshape(n_q, 1, BLK_Q, HD), (1, GRP, 1, 1)) \
        .reshape(n_q, QROWS, HD)
    r = jnp.arange(BLK_Q)[:, None]
    c = jnp.arange(BLK_K)[None, :]
    mask1 = jnp.where(r >= c, 0.0, NEG).astype(jnp.float32)
    mask = jnp.tile(mask1, (GRP, 1))

    attn = pl.pallas_call(
        _flash_kernel,
        grid=(B, NKV, n_q, n_k),
        in_specs=[
            pl.BlockSpec((GRP, BLK_Q, HD), lambda b, g, iq, ik: (g, b * n_q + iq, 0)),
            pl.BlockSpec((1, BLK_K, HD), lambda b, g, iq, ik: (g, b * n_k + ik, 0)),
            pl.BlockSpec((1, BLK_K, 128), lambda b, g, iq, ik: (g, b * n_k + ik, 0)),
            pl.BlockSpec((1, QROWS, HD), lambda b, g, iq, ik: (iq, 0, 0)),
            pl.BlockSpec((1, QROWS, HD), lambda b, g, iq, ik: (iq, 0, 0)),
            pl.BlockSpec((BLK_K, HD), lambda b, g, iq, ik: (ik, 0)),
            pl.BlockSpec((BLK_K, HD), lambda b, g, iq, ik: (ik, 0)),
            pl.BlockSpec((QROWS, BLK_K), lambda b, g, iq, ik: (0, 0)),
        ],
        out_specs=pl.BlockSpec((GRP, BLK_Q, HD),
                               lambda b, g, iq, ik: (g, b * n_q + iq, 0)),
        out_shape=jax.ShapeDtypeStruct((NH, T, HD), jnp.float32),
        scratch_shapes=[
            pltpu.VMEM((QROWS, 128), jnp.float32),
            pltpu.VMEM((QROWS, 128), jnp.float32),
            pltpu.VMEM((QROWS, 128), jnp.float32),
        ],
        compiler_params=pltpu.CompilerParams(
            dimension_semantics=("parallel", "parallel", "parallel", "arbitrary"),
        ),
    )(q, k, v, cos_q, sin_q, cos, sin, mask)

    res2, xn2, comb = pl.pallas_call(
        _post_kernel,
        grid=(n_t,),
        in_specs=[
            pl.BlockSpec((NH, BLK_T, HD), lambda i: (0, i, 0)),
            pl.BlockSpec((BLK_T, H), lambda i: (i, 0)),
            pl.BlockSpec((1, H), lambda i: (0, 0)),
            pl.BlockSpec((NH * HD, H), lambda i: (0, 0)),
            pl.BlockSpec((H, 128), lambda i: (0, 0)),
        ],
        out_specs=[
            pl.BlockSpec((BLK_T, H), lambda i: (i, 0)),
            pl.BlockSpec((BLK_T, H), lambda i: (i, 0)),
            pl.BlockSpec((BLK_T, 128), lambda i: (i, 0)),
        ],
        out_shape=[
            jax.ShapeDtypeStruct((T, H), jnp.float32),
            jax.ShapeDtypeStruct((T, H), jnp.bfloat16),
            jax.ShapeDtypeStruct((T, 128), jnp.float32),
        ],
    )(attn, hs, ln2, wo_t, rw_t)

    wg_t = Wg.transpose(0, 2, 1).astype(jnp.bfloat16)
    wu_t = Wu.transpose(0, 2, 1).astype(jnp.bfloat16)
    wd_t = Wd.transpose(0, 2, 1).astype(jnp.bfloat16)

    y = pl.pallas_call(
        _moe_kernel,
        grid=(n_t, E),
        in_specs=[
            pl.BlockSpec((BLK_T, H), lambda i, e: (i, 0)),
            pl.BlockSpec((BLK_T, 128), lambda i, e: (i, 0)),
            pl.BlockSpec((BLK_T, H), lambda i, e: (i, 0)),
            pl.BlockSpec((1, H, I), lambda i, e: (e, 0, 0)),
            pl.BlockSpec((1, H, I), lambda i, e: (e, 0, 0)),
            pl.BlockSpec((1, I, H), lambda i, e: (e, 0, 0)),
        ],
        out_specs=pl.BlockSpec((BLK_T, H), lambda i, e: (i, 0)),
        out_shape=jax.ShapeDtypeStruct((T, H), jnp.float32),
        compiler_params=pltpu.CompilerParams(
            dimension_semantics=("parallel", "arbitrary"),
        ),
    )(xn2, comb, res2, wg_t, wu_t, wd_t)

    return y.reshape(B, S, H)


@jax.jit
def kernel(hidden_states, cos, sin, Wq, Wk, Wv, Wo, ln1_w, ln2_w,
           router_w, Wg, Wu, Wd):
    return _build(hidden_states, cos, sin, Wq, Wk, Wv, Wo, ln1_w, ln2_w,
                  router_w, Wg, Wu, Wd)


# speedup vs baseline: 1.3554x; 1.3554x over previous
import jax
import jax.numpy as jnp
from jax.experimental import pallas as pl
from jax.experimental.pallas import tpu as pltpu

B, S, H = 2, 2048, 1024
NH, NKV, HD = 16, 4, 64
I, E, K = 512, 8, 2
GRP = NH // NKV
EPS = 1e-6
T = B * S

BLK_T = 512
BLK_Q = 512
BLK_K = 512
QROWS = GRP * BLK_Q
NEG = -1e30


def _rms(x, w):
    var = jnp.mean(x * x, axis=-1, keepdims=True)
    return x * jax.lax.rsqrt(var + EPS) * w


def _pre_kernel(hs_ref, ln1_ref, wq_ref, wk_ref, wv_ref, q_ref, k_ref, v_ref):
    x = hs_ref[...]
    xn = _rms(x, ln1_ref[...]).astype(jnp.bfloat16)
    q = jax.lax.dot(xn, wq_ref[...],
                    preferred_element_type=jnp.float32).astype(jnp.bfloat16)
    k = jax.lax.dot(xn, wk_ref[...],
                    preferred_element_type=jnp.float32).astype(jnp.bfloat16)
    v = jax.lax.dot(xn, wv_ref[...],
                    preferred_element_type=jnp.float32).astype(jnp.bfloat16)
    for h in range(NH):
        q_ref[h] = q[:, h * HD:(h + 1) * HD]
    for h in range(NKV):
        k_ref[h] = k[:, h * HD:(h + 1) * HD]
        v_ref[h] = v[:, h * HD:(h + 1) * HD]


def _rope(x, cos, sin):
    x1 = x[:, :HD // 2]
    x2 = x[:, HD // 2:]
    rot = jnp.concatenate([-x2, x1], axis=-1)
    return x * cos + rot * sin


def _flash_kernel(iq_ref, ik_ref, q_ref, k_ref, v_ref, cq_ref, sq_ref,
                  ck_ref, sk_ref, mask_ref, o_ref, acc_ref, m_ref, l_ref):
    p_id = pl.program_id(2)
    iq = iq_ref[p_id]
    ik = ik_ref[p_id]

    @pl.when(ik == 0)
    def _():
        m_ref[...] = jnp.full_like(m_ref, NEG)
        l_ref[...] = jnp.zeros_like(l_ref)
        acc_ref[...] = jnp.zeros_like(acc_ref)

    q = _rope(q_ref[...].reshape(QROWS, HD).astype(jnp.float32),
              cq_ref[0], sq_ref[0])
    k = _rope(k_ref[0].astype(jnp.float32), ck_ref[...], sk_ref[...])
    qb = (q * (HD ** -0.5)).astype(jnp.bfloat16)
    kb = k.astype(jnp.bfloat16)
    s = jax.lax.dot_general(qb, kb, (((1,), (1,)), ((), ())),
                            preferred_element_type=jnp.float32)
    ind = (ik == iq).astype(jnp.float32)
    s = s + mask_ref[...] * ind

    m_prev = m_ref[...]
    l_prev = l_ref[...]
    m_cur = jnp.max(s, axis=1, keepdims=True)
    m_next = jnp.maximum(m_prev, m_cur)
    alpha = jnp.exp(m_prev - m_next)
    p = jnp.exp(s - m_next[:, :1])
    l_ref[...] = alpha * l_prev + jnp.sum(p, axis=1, keepdims=True)
    m_ref[...] = m_next
    pv = jax.lax.dot(p.astype(jnp.bfloat16), v_ref[0],
                     preferred_element_type=jnp.float32)
    acc_ref[...] = acc_ref[...] * alpha[:, :1] + pv

    @pl.when(ik == iq)
    def _():
        out = acc_ref[...] / l_ref[:, :1]
        o_ref[...] = out.reshape(GRP, BLK_Q, HD)


def _post_kernel(attn_ref, hs_ref, ln2_ref, wo_ref, rw_ref,
                 res2_ref, xn2_ref, comb_ref):
    a = jnp.concatenate([attn_ref[h] for h in range(NH)], axis=1)
    o = jax.lax.dot(a.astype(jnp.bfloat16), wo_ref[...],
                    preferred_element_type=jnp.float32)
    res2 = hs_ref[...] + o
    res2_ref[...] = res2
    xn = _rms(res2, ln2_ref[...])
    xnb = xn.astype(jnp.bfloat16)
    xn2_ref[...] = xnb
    logits = jax.lax.dot(xnb, rw_ref[...], preferred_element_type=jnp.float32)
    lane = jax.lax.broadcasted_iota(jnp.int32, (BLK_T, 128), 1)
    valid = lane < E
    lg = jnp.where(valid, logits, NEG)
    mx = jnp.max(lg, axis=1, keepdims=True)
    ex = jnp.where(valid, jnp.exp(lg - mx), 0.0)
    denom = jnp.sum(ex, axis=1, keepdims=True)
    sc = ex / denom
    m1 = jnp.max(sc, axis=1, keepdims=True)
    idx1 = jnp.min(jnp.where(sc == m1, lane, 128), axis=1, keepdims=True)
    is1 = lane == idx1
    sc2 = jnp.where(is1, -1.0, sc)
    m2 = jnp.max(sc2, axis=1, keepdims=True)
    idx2 = jnp.min(jnp.where(sc2 == m2, lane, 128), axis=1, keepdims=True)
    is2 = lane == idx2
    wsum = m1 + m2
    comb_ref[...] = jnp.where(is1, m1 / wsum, jnp.where(is2, m2 / wsum, 0.0))


def _moe_kernel(x_ref, comb_ref, res2_ref, wg_ref, wu_ref, wd_ref, y_ref):
    e = pl.program_id(1)
    x = x_ref[...]
    gate = jax.lax.dot(x, wg_ref[0], preferred_element_type=jnp.float32)
    up = jax.lax.dot(x, wu_ref[0], preferred_element_type=jnp.float32)
    act = (gate * jax.lax.logistic(gate)) * up
    c = comb_ref[0][:, :1]
    actb = (act * c).astype(jnp.bfloat16)
    out_e = jax.lax.dot(actb, wd_ref[0], preferred_element_type=jnp.float32)

    @pl.when(e == 0)
    def _():
        y_ref[...] = res2_ref[...] + out_e

    @pl.when(e > 0)
    def _():
        y_ref[...] = y_ref[...] + out_e


def _build(hidden_states, cos, sin, Wq, Wk, Wv, Wo, ln1_w, ln2_w,
           router_w, Wg, Wu, Wd):
    hs = hidden_states.reshape(T, H)
    ln1 = ln1_w.reshape(1, H)
    ln2 = ln2_w.reshape(1, H)
    wq_t = Wq.T.astype(jnp.bfloat16)
    wk_t = Wk.T.astype(jnp.bfloat16)
    wv_t = Wv.T.astype(jnp.bfloat16)
    wo_t = Wo.T.astype(jnp.bfloat16)
    rw_pad = jnp.zeros((128, H), jnp.float32).at[:E].set(router_w)
    rw_t = rw_pad.T.astype(jnp.bfloat16)

    n_t = T // BLK_T
    q, k, v = pl.pallas_call(
        _pre_kernel,
        grid=(n_t,),
        in_specs=[
            pl.BlockSpec((BLK_T, H), lambda i: (i, 0)),
            pl.BlockSpec((1, H), lambda i: (0, 0)),
            pl.BlockSpec((H, NH * HD), lambda i: (0, 0)),
            pl.BlockSpec((H, NKV * HD), lambda i: (0, 0)),
            pl.BlockSpec((H, NKV * HD), lambda i: (0, 0)),
        ],
        out_specs=[
            pl.BlockSpec((NH, BLK_T, HD), lambda i: (0, i, 0)),
            pl.BlockSpec((NKV, BLK_T, HD), lambda i: (0, i, 0)),
            pl.BlockSpec((NKV, BLK_T, HD), lambda i: (0, i, 0)),
        ],
        out_shape=[
            jax.ShapeDtypeStruct((NH, T, HD), jnp.bfloat16),
            jax.ShapeDtypeStruct((NKV, T, HD), jnp.bfloat16),
            jax.ShapeDtypeStruct((NKV, T, HD), jnp.bfloat16),
        ],
    )(hs, ln1, wq_t, wk_t, wv_t)

    n_q = S // BLK_Q
    n_k = S // BLK_K
    cos_q = jnp.tile(cos.reshape(n_q, 1, BLK_Q, HD), (1, GRP, 1, 1)) \
        .reshape(n_q, QROWS, HD)
    sin_q = jnp.tile(sin.reshape(n_q, 1, BLK_Q, HD), (1, GRP, 1, 1)) \
        .reshape(n_q, QROWS, HD)
    r = jnp.arange(BLK_Q)[:, None]
    c = jnp.arange(BLK_K)[None, :]
    mask1 = jnp.where(r >= c, 0.0, NEG).astype(jnp.float32)
    mask = jnp.tile(mask1, (GRP, 1))

    pairs = [(a, b2) for a in range(n_q) for b2 in range(a + 1)]
    n_p = len(pairs)
    iq_arr = jnp.asarray([p[0] for p in pairs], jnp.int32)
    ik_arr = jnp.asarray([p[1] for p in pairs], jnp.int32)

    attn = pl.pallas_call(
        _flash_kernel,
        grid_spec=pltpu.PrefetchScalarGridSpec(
            num_scalar_prefetch=2,
            grid=(B, NKV, n_p),
            in_specs=[
                pl.BlockSpec((GRP, BLK_Q, HD),
                             lambda b, g, p, iqa, ika: (g, b * n_q + iqa[p], 0)),
                pl.BlockSpec((1, BLK_K, HD),
                             lambda b, g, p, iqa, ika: (g, b * n_k + ika[p], 0)),
                pl.BlockSpec((1, BLK_K, HD),
                             lambda b, g, p, iqa, ika: (g, b * n_k + ika[p], 0)),
                pl.BlockSpec((1, QROWS, HD),
                             lambda b, g, p, iqa, ika: (iqa[p], 0, 0)),
                pl.BlockSpec((1, QROWS, HD),
                             lambda b, g, p, iqa, ika: (iqa[p], 0, 0)),
                pl.BlockSpec((BLK_K, HD),
                             lambda b, g, p, iqa, ika: (ika[p], 0)),
                pl.BlockSpec((BLK_K, HD),
                             lambda b, g, p, iqa, ika: (ika[p], 0)),
                pl.BlockSpec((QROWS, BLK_K),
                             lambda b, g, p, iqa, ika: (0, 0)),
            ],
            out_specs=pl.BlockSpec(
                (GRP, BLK_Q, HD),
                lambda b, g, p, iqa, ika: (g, b * n_q + iqa[p], 0)),
            scratch_shapes=[
                pltpu.VMEM((QROWS, HD), jnp.float32),
                pltpu.VMEM((QROWS, 128), jnp.float32),
                pltpu.VMEM((QROWS, 128), jnp.float32),
            ],
        ),
        out_shape=jax.ShapeDtypeStruct((NH, T, HD), jnp.float32),
        compiler_params=pltpu.CompilerParams(
            dimension_semantics=("parallel", "parallel", "arbitrary"),
        ),
    )(iq_arr, ik_arr, q, k, v, cos_q, sin_q, cos, sin, mask)

    res2, xn2, comb = pl.pallas_call(
        _post_kernel,
        grid=(n_t,),
        in_specs=[
            pl.BlockSpec((NH, BLK_T, HD), lambda i: (0, i, 0)),
            pl.BlockSpec((BLK_T, H), lambda i: (i, 0)),
            pl.BlockSpec((1, H), lambda i: (0, 0)),
            pl.BlockSpec((NH * HD, H), lambda i: (0, 0)),
            pl.BlockSpec((H, 128), lambda i: (0, 0)),
        ],
        out_specs=[
            pl.BlockSpec((BLK_T, H), lambda i: (i, 0)),
            pl.BlockSpec((BLK_T, H), lambda i: (i, 0)),
            pl.BlockSpec((BLK_T, 128), lambda i: (i, 0)),
        ],
        out_shape=[
            jax.ShapeDtypeStruct((T, H), jnp.float32),
            jax.ShapeDtypeStruct((T, H), jnp.bfloat16),
            jax.ShapeDtypeStruct((T, 128), jnp.float32),
        ],
    )(attn, hs, ln2, wo_t, rw_t)

    comb_e = jnp.broadcast_to(comb.T[:E][:, :, None], (E, T, 1))
    comb_e = jnp.pad(comb_e, ((0, 0), (0, 0), (0, 127)))

    wg_t = Wg.transpose(0, 2, 1).astype(jnp.bfloat16)
    wu_t = Wu.transpose(0, 2, 1).astype(jnp.bfloat16)
    wd_t = Wd.transpose(0, 2, 1).astype(jnp.bfloat16)

    y = pl.pallas_call(
        _moe_kernel,
        grid=(n_t, E),
        in_specs=[
            pl.BlockSpec((BLK_T, H), lambda i, e: (i, 0)),
            pl.BlockSpec((1, BLK_T, 128), lambda i, e: (e, i, 0)),
            pl.BlockSpec((BLK_T, H), lambda i, e: (i, 0)),
            pl.BlockSpec((1, H, I), lambda i, e: (e, 0, 0)),
            pl.BlockSpec((1, H, I), lambda i, e: (e, 0, 0)),
            pl.BlockSpec((1, I, H), lambda i, e: (e, 0, 0)),
        ],
        out_specs=pl.BlockSpec((BLK_T, H), lambda i, e: (i, 0)),
        out_shape=jax.ShapeDtypeStruct((T, H), jnp.float32),
        compiler_params=pltpu.CompilerParams(
            dimension_semantics=("parallel", "arbitrary"),
        ),
    )(xn2, comb_e, res2, wg_t, wu_t, wd_t)

    return y.reshape(B, S, H)


@jax.jit
def kernel(hidden_states, cos, sin, Wq, Wk, Wv, Wo, ln1_w, ln2_w,
           router_w, Wg, Wu, Wd):
    return _build(hidden_states, cos, sin, Wq, Wk, Wv, Wo, ln1_w, ln2_w,
                  router_w, Wg, Wu, Wd)
